# baseline (speedup 1.0000x reference)
import functools

import jax
import jax.numpy as jnp
from jax import lax
from jax.experimental import pallas as pl
from jax.experimental.pallas import tpu as pltpu
from jax.experimental.pallas import tpu_sc as plsc

_N = 10000
_D = 128
_E = 320000
_C = 128
_CPT = 158
_EPT = _CPT * _C
_EPAD = 16 * _EPT
_PW = 3 * _C
_CAP = 12288
_NCH = _CAP // _C
_AR = 10240
_ZR = _AR // 16
_R = 1000
_G = _N // _R


def _lrelu(v):
    return jnp.where(v >= 0, v, 0.01 * v)


def _dot(a, b):
    return jnp.dot(a, b, preferred_element_type=jnp.float32)


def _mlp_in(np8, wn, bn, wi, bi):
    def body(np_r, wn_r, bn_r, wi_r, bi_r, o_r):
        h = _lrelu(_dot(np_r[...], wn_r[...]) + bn_r[...])
        o_r[...] = _lrelu(_dot(h, wi_r[...]) + bi_r[...])

    return pl.pallas_call(
        body,
        grid=(_G,),
        in_specs=[
            pl.BlockSpec((_R, 8), lambda i: (i, 0)),
            pl.BlockSpec((8, _D), lambda i: (0, 0)),
            pl.BlockSpec((1, _D), lambda i: (0, 0)),
            pl.BlockSpec((_D, _D), lambda i: (0, 0)),
            pl.BlockSpec((1, _D), lambda i: (0, 0)),
        ],
        out_specs=pl.BlockSpec((_R, _D), lambda i: (i, 0)),
        out_shape=jax.ShapeDtypeStruct((_N, _D), jnp.float32),
    )(np8, wn, bn, wi, bi)


def _combine(x, s0, s1, cnt0, cnt1, wroot, wrel, b, tail_args=None):
    tail = tail_args is not None

    def body(x_r, s0_r, s1_r, c0_r, c1_r, wroot_r, wrel_r, b_r, *rest):
        if tail:
            wo1_r, bo1_r, wo2_r, bo2_r, o_r = rest
        else:
            (o_r,) = rest
        wrel_v = wrel_r[...]
        inv0 = 1.0 / jnp.maximum(c0_r[...][:, 0:1], 1.0)
        inv1 = 1.0 / jnp.maximum(c1_r[...][:, 0:1], 1.0)
        y = _dot(x_r[...], wroot_r[...]) + b_r[...]
        y = y + _dot(s0_r[...] * inv0, wrel_v[0])
        y = y + _dot(s1_r[...] * inv1, wrel_v[1])
        if tail:
            z = _lrelu(_dot(y, wo1_r[...]) + bo1_r[...])
            y = _dot(z, wo2_r[...]) + bo2_r[...]
        o_r[...] = y

    def full(shape):
        return pl.BlockSpec(shape, lambda i: tuple(0 for _ in shape))

    in_specs = [
        pl.BlockSpec((_R, _D), lambda i: (i, 0)),
        pl.BlockSpec((_R, _D), lambda i: (i, 0)),
        pl.BlockSpec((_R, _D), lambda i: (i, 0)),
        pl.BlockSpec((_R, 16), lambda i: (i, 0)),
        pl.BlockSpec((_R, 16), lambda i: (i, 0)),
        full((_D, _D)),
        full((2, _D, _D)),
        full((1, _D)),
    ]
    args = [x, s0, s1, cnt0, cnt1, wroot, wrel, b]
    if tail:
        in_specs += [full((_D, _D)), full((1, _D)),
                     full((_D, _D)), full((1, _D))]
        args += list(tail_args)
    return pl.pallas_call(
        body,
        grid=(_G,),
        in_specs=in_specs,
        out_specs=pl.BlockSpec((_R, _D), lambda i: (i, 0)),
        out_shape=jax.ShapeDtypeStruct((_N, _D), jnp.float32),
    )(*args)


def _sc_partition(pack, tcap):
    mesh = plsc.VectorSubcoreMesh(core_axis_name="c", subcore_axis_name="s")
    f32, i32 = jnp.float32, jnp.int32

    @functools.partial(
        pl.kernel, mesh=mesh,
        out_type=jax.ShapeDtypeStruct((32 * _CAP + 2 * _C,), i32),
        scratch_types=[
            pltpu.VMEM((_PW,), i32), pltpu.VMEM((_PW,), i32),
            pltpu.VMEM((_CAP,), i32),
            pltpu.SemaphoreType.DMA, pltpu.SemaphoreType.DMA,
        ],
        compiler_params=pltpu.CompilerParams(needs_layout_passes=False),
    )
    def k(pack_h, tcap_h, pedge_h, pk0, pk1, ced, isem0, isem1):
        c = lax.axis_index("c")
        s = lax.axis_index("s")
        pk = (pk0, pk1)
        isem = (isem0, isem1)
        pltpu.sync_copy(tcap_h, ced)

        def cbase(i):
            return pl.multiple_of((s * _CPT + i) * _PW, _C)

        for b in range(2):
            pltpu.async_copy(pack_h.at[pl.ds(cbase(b), _PW)], pk[b],
                             isem[b])

        lanes = lax.iota(i32, 16)
        trash = jnp.full((16,), _N, i32)

        def body(g, pos):
            for b in range(2):
                pltpu.make_async_copy(pack_h.at[pl.ds(0, _PW)], pk[b],
                                      isem[b]).wait()
                for j in range(_C // 16):
                    s16 = pk[b][pl.ds(j * 16, 16)]
                    d16 = pk[b][pl.ds(_C + j * 16, 16)]
                    t16 = pk[b][pl.ds(2 * _C + j * 16, 16)]
                    key = jnp.where(t16 == c, jnp.int32(1), jnp.int32(0))
                    val = s16 * 16384 + d16
                    _, vv = plsc.sort_key_val(key, val, descending=True)
                    off = jnp.minimum(pos, _CAP - 16)
                    plsc.store_scatter(ced, [lanes + off], vv)
                    pos = pos + plsc.all_reduce_population_count(
                        t16 == c)
                pltpu.async_copy(
                    pack_h.at[pl.ds(cbase(2 * g + 2 + b), _PW)], pk[b],
                    isem[b])
            return pos

        pos = lax.fori_loop(0, _CPT // 2, body,
                            jnp.zeros((16,), i32))
        plsc.store_scatter(ced, [lanes + jnp.minimum(pos, _CAP - 16)],
                           trash)
        for b in range(2):
            pltpu.make_async_copy(pack_h.at[pl.ds(0, _PW)], pk[b],
                                  isem[b]).wait()
        tb = pl.multiple_of((c * 16 + s) * _CAP, _C)
        pltpu.sync_copy(ced, pedge_h.at[pl.ds(tb, _CAP)])

    return k(pack, tcap)


def _sc_segsum(x, pedge, zrows, zrow1, ones1, with_counts):
    mesh = plsc.VectorSubcoreMesh(core_axis_name="c", subcore_axis_name="s")
    f32, i32 = jnp.float32, jnp.int32

    out_type = [
        jax.ShapeDtypeStruct((_AR, _D), f32),
        jax.ShapeDtypeStruct((_AR, _D), f32),
    ]
    scratch = (
        [pltpu.VMEM((_C,), i32)] * 4
        + [pltpu.VMEM((_C,), i32)] * 2
        + [pltpu.VMEM((_C, _D), f32)] * 2
        + [pltpu.VMEM_SHARED((_AR, _D), f32)]
        + [pltpu.SemaphoreType.DMA] * 4
    )
    if with_counts:
        out_type += [jax.ShapeDtypeStruct((_AR,), f32),
                     jax.ShapeDtypeStruct((_AR,), f32)]
        scratch += [pltpu.VMEM((_C,), f32),
                    pltpu.VMEM_SHARED((_AR,), f32)]

    @functools.partial(pl.kernel, mesh=mesh, out_type=out_type,
                       scratch_types=scratch)
    def k(*refs):
        if with_counts:
            (x_h, ped_h, z_h, z1_h, ones_h,
             s0_h, s1_h, cnt0_h, cnt1_h,
             ev0, ev1, sv0, sv1, dv0, dv1, rows0, rows1, acc,
             isem0, isem1, gsem0, gsem1, ones_v, cacc) = refs
        else:
            (x_h, ped_h, z_h, s0_h, s1_h,
             ev0, ev1, sv0, sv1, dv0, dv1, rows0, rows1, acc,
             isem0, isem1, gsem0, gsem1) = refs
        ev = (ev0, ev1)
        sv = (sv0, sv1)
        dv = (dv0, dv1)
        rows = (rows0, rows1)
        isem = (isem0, isem1)
        gsem = (gsem0, gsem1)
        c = lax.axis_index("c")
        s = lax.axis_index("s")
        pltpu.sync_copy(z_h, acc.at[pl.ds(s * _ZR, _ZR)])
        if with_counts:
            pltpu.sync_copy(z1_h, cacc.at[pl.ds(s * _ZR, _ZR)])
            pltpu.sync_copy(ones_h, ones_v)
        plsc.subcore_barrier()

        tb = pl.multiple_of((c * 16 + s) * _CAP, _C)

        def fetch(i, b):
            base = pl.multiple_of(tb + i * _C, _C)
            pltpu.async_copy(ped_h.at[pl.ds(base, _C)], ev[b], isem[b])

        def stage(b):
            for j in range(_C // 16):
                sl = pl.ds(j * 16, 16)
                e16 = ev[b][sl]
                sv[b][sl] = lax.shift_right_logical(e16, 14)
                dv[b][sl] = jnp.bitwise_and(e16, 16383)

        def wait_idx(b):
            pltpu.make_async_copy(ped_h.at[pl.ds(0, _C)], ev[b],
                                  isem[b]).wait()

        fetch(0, 0)
        fetch(1, 1)

        def body(g, carry):
            i0 = 2 * g
            wait_idx(0)
            stage(0)
            h0 = pltpu.async_copy(x_h.at[sv0], rows0, gsem0)
            if with_counts:
                pltpu.sync_copy(ones_v, cacc.at[dv0], add=True)
            wait_idx(1)
            stage(1)
            h1 = pltpu.async_copy(x_h.at[sv1], rows1, gsem1)
            if with_counts:
                pltpu.sync_copy(ones_v, cacc.at[dv1], add=True)
            h0.wait()
            pltpu.sync_copy(rows0, acc.at[dv0], add=True)
            fetch(i0 + 2, 0)
            h1.wait()
            pltpu.sync_copy(rows1, acc.at[dv1], add=True)
            fetch(i0 + 3, 1)
            return carry

        lax.fori_loop(0, _NCH // 2, body, 0)
        for b in range(2):
            wait_idx(b)
        plsc.subcore_barrier()

        @pl.when(c == 0)
        def _():
            pltpu.sync_copy(acc.at[pl.ds(s * _ZR, _ZR)],
                            s0_h.at[pl.ds(s * _ZR, _ZR)])
            if with_counts:
                pltpu.sync_copy(cacc.at[pl.ds(s * _ZR, _ZR)],
                                cnt0_h.at[pl.ds(s * _ZR, _ZR)])

        @pl.when(c == 1)
        def _():
            pltpu.sync_copy(acc.at[pl.ds(s * _ZR, _ZR)],
                            s1_h.at[pl.ds(s * _ZR, _ZR)])
            if with_counts:
                pltpu.sync_copy(cacc.at[pl.ds(s * _ZR, _ZR)],
                                cnt1_h.at[pl.ds(s * _ZR, _ZR)])

    if with_counts:
        return k(x, pedge, zrows, zrow1, ones1)
    return k(x, pedge, zrows)


def kernel(des, tweet, num_prop, cat_prop, W_num, b_num, W_in, b_in, W_rel,
           W_root, b_rgcn, W_out1, b_out1, W_out2, b_out2, edge_index,
           edge_type):
    f32 = jnp.float32
    np8 = jnp.pad(num_prop, ((0, 0), (0, 2)))
    wn8 = jnp.pad(W_num, ((0, 2), (0, 0)))
    bn = b_num.reshape(1, _D)
    bi = b_in.reshape(1, _D)
    br = b_rgcn.reshape(1, _D)
    bo1 = b_out1.reshape(1, _D)
    wo2 = jnp.pad(W_out2, ((0, 0), (0, _D - 2)))
    bo2 = jnp.pad(b_out2, (0, _D - 2)).reshape(1, _D)

    pad = _EPAD - _E
    srcp = jnp.concatenate([edge_index[0], jnp.zeros((pad,), jnp.int32)])
    dstp = jnp.concatenate([edge_index[1], jnp.zeros((pad,), jnp.int32)])
    typp = jnp.concatenate([edge_type, jnp.full((pad,), 2, jnp.int32)])
    pack = jnp.stack([srcp.reshape(-1, _C), dstp.reshape(-1, _C),
                      typp.reshape(-1, _C)], axis=1).reshape(-1)
    pack = jnp.concatenate([pack, jnp.zeros((4 * _PW,), jnp.int32)])
    zrows = jnp.zeros((_ZR, _D), f32)
    zrow1 = jnp.zeros((_ZR,), f32)
    ones1 = jnp.ones((_C,), f32)

    tcap = jnp.full((_CAP,), _N, jnp.int32)
    pedge = _sc_partition(pack, tcap)

    x0 = _mlp_in(np8, wn8, bn, W_in, bi)
    s0a, s1a, c0f, c1f = _sc_segsum(x0, pedge, zrows, zrow1, ones1,
                                    with_counts=True)
    cnt0 = jnp.broadcast_to(c0f[:, None], (_AR, 16))
    cnt1 = jnp.broadcast_to(c1f[:, None], (_AR, 16))
    x1 = _combine(x0, s0a, s1a, cnt0, cnt1, W_root, W_rel, br)
    s0b, s1b = _sc_segsum(x1, pedge, zrows, zrow1, ones1,
                          with_counts=False)
    out = _combine(x1, s0b, s1b, cnt0, cnt1, W_root, W_rel, br,
                   tail_args=(W_out1, bo1, wo2, bo2))
    return out[:, :2]


# speedup vs baseline: 4.9709x; 4.9709x over previous
import functools

import jax
import jax.numpy as jnp
from jax import lax
from jax.experimental import pallas as pl
from jax.experimental.pallas import tpu as pltpu
from jax.experimental.pallas import tpu_sc as plsc

_N = 10000
_D = 128
_E = 320000
_C = 128
_CPT = 158
_EPT = _CPT * _C
_EPAD = 16 * _EPT
_PW = 3 * _C
_AR = 10240
_ZR = _AR // 16
_R = 1000
_G = _N // _R


def _lrelu(v):
    return jnp.where(v >= 0, v, 0.01 * v)


def _dot(a, b):
    return jnp.dot(a, b, preferred_element_type=jnp.float32)


def _mlp_in(np8, wn, bn, wi, bi):
    def body(np_r, wn_r, bn_r, wi_r, bi_r, o_r):
        h = _lrelu(_dot(np_r[...], wn_r[...]) + bn_r[...])
        o_r[...] = _lrelu(_dot(h, wi_r[...]) + bi_r[...])

    return pl.pallas_call(
        body,
        grid=(_G,),
        in_specs=[
            pl.BlockSpec((_R, 8), lambda i: (i, 0)),
            pl.BlockSpec((8, _D), lambda i: (0, 0)),
            pl.BlockSpec((1, _D), lambda i: (0, 0)),
            pl.BlockSpec((_D, _D), lambda i: (0, 0)),
            pl.BlockSpec((1, _D), lambda i: (0, 0)),
        ],
        out_specs=pl.BlockSpec((_R, _D), lambda i: (i, 0)),
        out_shape=jax.ShapeDtypeStruct((_N, _D), jnp.float32),
    )(np8, wn, bn, wi, bi)


def _combine(x, s0, s1, cnt0, cnt1, wroot, wrel, b, tail_args=None):
    tail = tail_args is not None

    def body(x_r, s0_r, s1_r, c0_r, c1_r, wroot_r, wrel_r, b_r, *rest):
        if tail:
            wo1_r, bo1_r, wo2_r, bo2_r, o_r = rest
        else:
            (o_r,) = rest
        wrel_v = wrel_r[...]
        inv0 = 1.0 / jnp.maximum(c0_r[...][:, 0:1], 1.0)
        inv1 = 1.0 / jnp.maximum(c1_r[...][:, 0:1], 1.0)
        y = _dot(x_r[...], wroot_r[...]) + b_r[...]
        y = y + _dot(s0_r[...] * inv0, wrel_v[0])
        y = y + _dot(s1_r[...] * inv1, wrel_v[1])
        if tail:
            z = _lrelu(_dot(y, wo1_r[...]) + bo1_r[...])
            y = _dot(z, wo2_r[...]) + bo2_r[...]
        o_r[...] = y

    def full(shape):
        return pl.BlockSpec(shape, lambda i: tuple(0 for _ in shape))

    in_specs = [
        pl.BlockSpec((_R, _D), lambda i: (i, 0)),
        pl.BlockSpec((_R, _D), lambda i: (i, 0)),
        pl.BlockSpec((_R, _D), lambda i: (i, 0)),
        pl.BlockSpec((_R, 16), lambda i: (i, 0)),
        pl.BlockSpec((_R, 16), lambda i: (i, 0)),
        full((_D, _D)),
        full((2, _D, _D)),
        full((1, _D)),
    ]
    args = [x, s0, s1, cnt0, cnt1, wroot, wrel, b]
    if tail:
        in_specs += [full((_D, _D)), full((1, _D)),
                     full((_D, _D)), full((1, _D))]
        args += list(tail_args)
    return pl.pallas_call(
        body,
        grid=(_G,),
        in_specs=in_specs,
        out_specs=pl.BlockSpec((_R, _D), lambda i: (i, 0)),
        out_shape=jax.ShapeDtypeStruct((_N, _D), jnp.float32),
    )(*args)


def _sc_segsum(x, pack, zrows, zrow1, ones1, with_counts):
    mesh = plsc.VectorSubcoreMesh(core_axis_name="c", subcore_axis_name="s")
    f32, i32 = jnp.float32, jnp.int32

    out_type = [
        jax.ShapeDtypeStruct((_AR, _D), f32),
        jax.ShapeDtypeStruct((_AR, _D), f32),
    ]
    scratch = [
        pltpu.VMEM((_PW,), i32), pltpu.VMEM((_PW,), i32),
        pltpu.VMEM((_C,), i32), pltpu.VMEM((_C,), i32),
        pltpu.VMEM((_C,), i32), pltpu.VMEM((_C,), i32),
        pltpu.VMEM((_C, _D), f32), pltpu.VMEM((_C, _D), f32),
        pltpu.VMEM_SHARED((_AR, _D), f32),
        pltpu.SemaphoreType.DMA, pltpu.SemaphoreType.DMA,
        pltpu.SemaphoreType.DMA, pltpu.SemaphoreType.DMA,
    ]
    if with_counts:
        out_type += [jax.ShapeDtypeStruct((_AR,), f32),
                     jax.ShapeDtypeStruct((_AR,), f32)]
        scratch += [pltpu.VMEM((_C,), f32),
                    pltpu.VMEM_SHARED((_AR,), f32)]

    @functools.partial(pl.kernel, mesh=mesh, out_type=out_type,
                       scratch_types=scratch)
    def k(*refs):
        if with_counts:
            (x_h, pack_h, z_h, z1_h, ones_h, s0_h, s1_h, cnt0_h, cnt1_h,
             pk0, pk1, gix0, gix1, eff0, eff1, rows0, rows1, acc,
             isem0, isem1, gsem0, gsem1, ones_v, cacc) = refs
        else:
            (x_h, pack_h, z_h, s0_h, s1_h,
             pk0, pk1, gix0, gix1, eff0, eff1, rows0, rows1, acc,
             isem0, isem1, gsem0, gsem1) = refs
        c = lax.axis_index("c")
        s = lax.axis_index("s")
        pltpu.sync_copy(z_h, acc.at[pl.ds(s * _ZR, _ZR)])
        if with_counts:
            pltpu.sync_copy(z1_h, cacc.at[pl.ds(s * _ZR, _ZR)])
            pltpu.sync_copy(ones_h, ones_v)
        plsc.subcore_barrier()

        def cbase(i):
            return pl.multiple_of((s * _CPT + i) * _PW, _C)

        pltpu.async_copy(pack_h.at[pl.ds(cbase(0), _PW)], pk0, isem0)
        pltpu.async_copy(pack_h.at[pl.ds(cbase(1), _PW)], pk1, isem1)

        def stage(pk, gix, eff):
            for j in range(_C // 16):
                sl = pl.ds(j * 16, 16)
                gix[sl] = pk[pl.ds(j * 16, 16)]
                eff[sl] = jnp.where(pk[pl.ds(2 * _C + j * 16, 16)] == c,
                                    pk[pl.ds(_C + j * 16, 16)], _N)

        def body(g, carry):
            i0 = 2 * g
            pltpu.make_async_copy(pack_h.at[pl.ds(0, _PW)], pk0,
                                  isem0).wait()
            stage(pk0, gix0, eff0)
            h0 = pltpu.async_copy(x_h.at[gix0], rows0, gsem0)
            pltpu.make_async_copy(pack_h.at[pl.ds(0, _PW)], pk1,
                                  isem1).wait()
            stage(pk1, gix1, eff1)
            h1 = pltpu.async_copy(x_h.at[gix1], rows1, gsem1)
            h0.wait()
            pltpu.sync_copy(rows0, acc.at[eff0], add=True)
            if with_counts:
                pltpu.sync_copy(ones_v, cacc.at[eff0], add=True)
            pltpu.async_copy(pack_h.at[pl.ds(cbase(i0 + 2), _PW)], pk0,
                             isem0)
            h1.wait()
            pltpu.sync_copy(rows1, acc.at[eff1], add=True)
            if with_counts:
                pltpu.sync_copy(ones_v, cacc.at[eff1], add=True)
            pltpu.async_copy(pack_h.at[pl.ds(cbase(i0 + 3), _PW)], pk1,
                             isem1)
            return carry

        lax.fori_loop(0, _CPT // 2, body, 0)
        pltpu.make_async_copy(pack_h.at[pl.ds(0, _PW)], pk0, isem0).wait()
        pltpu.make_async_copy(pack_h.at[pl.ds(0, _PW)], pk1, isem1).wait()
        plsc.subcore_barrier()

        @pl.when(c == 0)
        def _():
            pltpu.sync_copy(acc.at[pl.ds(s * _ZR, _ZR)],
                            s0_h.at[pl.ds(s * _ZR, _ZR)])
            if with_counts:
                pltpu.sync_copy(cacc.at[pl.ds(s * _ZR, _ZR)],
                                cnt0_h.at[pl.ds(s * _ZR, _ZR)])

        @pl.when(c == 1)
        def _():
            pltpu.sync_copy(acc.at[pl.ds(s * _ZR, _ZR)],
                            s1_h.at[pl.ds(s * _ZR, _ZR)])
            if with_counts:
                pltpu.sync_copy(cacc.at[pl.ds(s * _ZR, _ZR)],
                                cnt1_h.at[pl.ds(s * _ZR, _ZR)])

    if with_counts:
        return k(x, pack, zrows, zrow1, ones1)
    return k(x, pack, zrows)


def kernel(des, tweet, num_prop, cat_prop, W_num, b_num, W_in, b_in, W_rel,
           W_root, b_rgcn, W_out1, b_out1, W_out2, b_out2, edge_index,
           edge_type):
    f32 = jnp.float32
    np8 = jnp.pad(num_prop, ((0, 0), (0, 2)))
    wn8 = jnp.pad(W_num, ((0, 2), (0, 0)))
    bn = b_num.reshape(1, _D)
    bi = b_in.reshape(1, _D)
    br = b_rgcn.reshape(1, _D)
    bo1 = b_out1.reshape(1, _D)
    wo2 = jnp.pad(W_out2, ((0, 0), (0, _D - 2)))
    bo2 = jnp.pad(b_out2, (0, _D - 2)).reshape(1, _D)

    pad = _EPAD - _E
    srcp = jnp.concatenate([edge_index[0], jnp.zeros((pad,), jnp.int32)])
    dstp = jnp.concatenate([edge_index[1], jnp.zeros((pad,), jnp.int32)])
    typp = jnp.concatenate([edge_type, jnp.full((pad,), 2, jnp.int32)])
    pack = jnp.stack([srcp.reshape(-1, _C), dstp.reshape(-1, _C),
                      typp.reshape(-1, _C)], axis=1).reshape(-1)
    pack = jnp.concatenate([pack, jnp.zeros((2 * _PW,), jnp.int32)])
    zrows = jnp.zeros((_ZR, _D), f32)
    zrow1 = jnp.zeros((_ZR,), f32)
    ones1 = jnp.ones((_C,), f32)

    x0 = _mlp_in(np8, wn8, bn, W_in, bi)
    s0a, s1a, c0f, c1f = _sc_segsum(x0, pack, zrows, zrow1, ones1,
                                    with_counts=True)
    cnt0 = jnp.broadcast_to(c0f[:, None], (_AR, 16))
    cnt1 = jnp.broadcast_to(c1f[:, None], (_AR, 16))
    x1 = _combine(x0, s0a, s1a, cnt0, cnt1, W_root, W_rel, br)
    s0b, s1b = _sc_segsum(x1, pack, zrows, zrow1, ones1,
                          with_counts=False)
    out = _combine(x1, s0b, s1b, cnt0, cnt1, W_root, W_rel, br,
                   tail_args=(W_out1, bo1, wo2, bo2))
    return out[:, :2]
